# baseline (speedup 1.0000x reference)
import jax
import jax.numpy as jnp
from jax.experimental import pallas as pl
from jax.experimental.pallas import tpu as pltpu

N = 4096
CUTOFF_SQ = 0.25 * 0.25
BR = 512


def _pair_kernel(xi_ref, xt_ref, wb_ref, out_ref):
    w = wb_ref[0, 0]
    b = wb_ref[0, 1]
    dsq = jnp.zeros((BR, N), jnp.float32)
    for k in range(3):
        d = xt_ref[k:k + 1, :] - xi_ref[:, k:k + 1]
        off = (d < -0.5).astype(jnp.float32) - (d >= 0.5).astype(jnp.float32)
        d = d + off
        dsq = dsq + d * d
    mask = (dsq < CUTOFF_SQ) & (dsq != 0.0)
    r = jnp.sqrt(dsq)
    out_ref[...] = jnp.where(mask, r * w + b, 0.0)


def kernel(xyz, W, b):
    xt = xyz.T
    wb = jnp.concatenate([W.reshape(1, 1), b.reshape(1, 1)], axis=1)
    out = pl.pallas_call(
        _pair_kernel,
        grid=(N // BR,),
        in_specs=[
            pl.BlockSpec((BR, 3), lambda i: (i, 0)),
            pl.BlockSpec((3, N), lambda i: (0, 0)),
            pl.BlockSpec((1, 2), lambda i: (0, 0)),
        ],
        out_specs=pl.BlockSpec((BR, N), lambda i: (i, 0)),
        out_shape=jax.ShapeDtypeStruct((N, N), jnp.float32),
    )(xyz, xt, wb)
    return out[..., None]


# speedup vs baseline: 1.2071x; 1.2071x over previous
import jax
import jax.numpy as jnp
from jax.experimental import pallas as pl
from jax.experimental.pallas import tpu as pltpu

N = 4096
CUTOFF_SQ = 0.25 * 0.25
BR = 512


def _pair_kernel(xi_ref, xt_ref, wb_ref, out_ref):
    w = wb_ref[0, 0]
    b = wb_ref[0, 1]
    dsq = jnp.zeros((BR, N), jnp.float32)
    for k in range(3):
        d = xt_ref[k:k + 1, :] - xi_ref[:, k:k + 1]
        a = jnp.abs(d)
        m = jnp.minimum(a, 1.0 - a)
        dsq = dsq + m * m
    mask = (dsq < CUTOFF_SQ) & (dsq != 0.0)
    r = jnp.sqrt(dsq)
    out_ref[...] = jnp.where(mask, r * w + b, 0.0)


def kernel(xyz, W, b):
    xt = xyz.T
    wb = jnp.concatenate([W.reshape(1, 1), b.reshape(1, 1)], axis=1)
    out = pl.pallas_call(
        _pair_kernel,
        grid=(N // BR,),
        in_specs=[
            pl.BlockSpec((BR, 3), lambda i: (i, 0)),
            pl.BlockSpec((3, N), lambda i: (0, 0)),
            pl.BlockSpec((1, 2), lambda i: (0, 0)),
        ],
        out_specs=pl.BlockSpec((BR, N), lambda i: (i, 0)),
        out_shape=jax.ShapeDtypeStruct((N, N), jnp.float32),
    )(xyz, xt, wb)
    return out[..., None]
